# baseline (speedup 1.0000x reference)
import jax
import jax.numpy as jnp
from jax import lax
from jax.experimental import pallas as pl
from jax.experimental.pallas import tpu as pltpu
from jax.experimental.pallas import tpu_sc as plsc

N_NODES = 10000
N_EDGES = 320000
D = 128

NC = 2
NS = 16
NW = NC * NS
L = 16
CHUNK = 32
CPW = 320
IB = 16
EPW = CPW * CHUNK
E_PAD = NW * EPW
N_ACC = 10240
RPS = N_ACC // NS
BLK_E = IB * CHUNK


def _sc_body(x_hbm, row_hbm, col_hbm, colf_hbm, zeros_hbm,
             acc_out, deg_out,
             row_v, col_v, colf_v, rows_v, deg_local,
             acc_s, sem):
    c = lax.axis_index("c")
    s = lax.axis_index("s")
    wid = s * NC + c

    pltpu.sync_copy(zeros_hbm, rows_v.at[0])

    for k in range(RPS // CHUNK):
        off = s * RPS + k * CHUNK
        pltpu.sync_copy(rows_v.at[0], acc_s.at[pl.ds(off, CHUNK)])

    zeros16 = jnp.zeros((L,), jnp.float32)

    @pl.loop(0, N_ACC // L)
    def _zero_deg(i):
        deg_local[pl.ds(i * L, L)] = zeros16

    plsc.subcore_barrier()

    ones16 = jnp.full((L,), 1.0, jnp.float32)

    for ib in range(CPW // IB):
        pltpu.sync_copy(row_hbm.at[wid, pl.ds(ib * IB, IB)], row_v)
        pltpu.sync_copy(col_hbm.at[wid, pl.ds(ib * IB, IB)], col_v)
        pltpu.sync_copy(colf_hbm.at[pl.ds(wid * EPW + ib * BLK_E, BLK_E)],
                        colf_v)

        pltpu.async_copy(x_hbm.at[row_v.at[0]], rows_v.at[0], sem)

        @pl.loop(0, IB - 1)
        def _edge_chunk(j):
            slot = lax.rem(j, 2)
            nslot = lax.rem(j + 1, 2)
            pltpu.make_async_copy(x_hbm.at[row_v.at[j]],
                                  rows_v.at[slot], sem).wait()
            pltpu.async_copy(x_hbm.at[row_v.at[j + 1]], rows_v.at[nslot], sem)
            pltpu.sync_copy(rows_v.at[slot], acc_s.at[col_v.at[j]], add=True)

        lastslot = (IB - 1) % 2
        pltpu.make_async_copy(x_hbm.at[row_v.at[IB - 1]],
                              rows_v.at[lastslot], sem).wait()
        pltpu.sync_copy(rows_v.at[lastslot], acc_s.at[col_v.at[IB - 1]],
                        add=True)

        @pl.loop(0, BLK_E // L)
        def _deg(j):
            idx = colf_v[pl.ds(j * L, L)]
            plsc.addupdate_scatter(deg_local, [idx], ones16)

    plsc.subcore_barrier()

    out_off = s * RPS
    pltpu.sync_copy(acc_s.at[pl.ds(out_off, RPS)],
                    acc_out.at[c, pl.ds(out_off, RPS)])
    pltpu.sync_copy(deg_local, deg_out.at[pl.ds(wid * N_ACC, N_ACC)])


def _sc_aggregate(x, row_p, col_p, col_flat, zeros):
    mesh = plsc.VectorSubcoreMesh(core_axis_name="c", subcore_axis_name="s")
    return pl.kernel(
        _sc_body,
        out_type=(
            jax.ShapeDtypeStruct((NC, N_ACC, D), jnp.float32),
            jax.ShapeDtypeStruct((NW * N_ACC,), jnp.float32),
        ),
        mesh=mesh,
        compiler_params=pltpu.CompilerParams(needs_layout_passes=False),
        scratch_types=[
            pltpu.VMEM((IB, CHUNK), jnp.int32),
            pltpu.VMEM((IB, CHUNK), jnp.int32),
            pltpu.VMEM((BLK_E,), jnp.int32),
            pltpu.VMEM((2, CHUNK, D), jnp.float32),
            pltpu.VMEM((N_ACC,), jnp.float32),
            pltpu.VMEM_SHARED((N_ACC, D), jnp.float32),
            pltpu.SemaphoreType.DMA,
        ],
    )(x, row_p, col_p, col_flat, zeros)


BR = 1024


def _tc_body(acc_ref, deg_ref, w_ref, b_ref, o_ref):
    ssum = acc_ref[0] + acc_ref[1]
    dcol = jnp.sum(deg_ref[...], axis=0)[:, None]
    r = ssum / jnp.maximum(dcol, 1.0)
    o_ref[...] = jnp.dot(r, w_ref[...],
                         preferred_element_type=jnp.float32,
                         precision=jax.lax.Precision.HIGHEST) + b_ref[...]


def _tc_project(acc, deg, W, b2):
    return pl.pallas_call(
        _tc_body,
        grid=(N_ACC // BR,),
        in_specs=[
            pl.BlockSpec((NC, BR, D), lambda i: (0, i, 0)),
            pl.BlockSpec((NW, BR), lambda i: (0, i)),
            pl.BlockSpec((D, D), lambda i: (0, 0)),
            pl.BlockSpec((1, D), lambda i: (0, 0)),
        ],
        out_specs=pl.BlockSpec((BR, D), lambda i: (i, 0)),
        out_shape=jax.ShapeDtypeStruct((N_ACC, D), jnp.float32),
    )(acc, deg, W, b2)


@jax.jit
def kernel(x, edge_index, W, b):
    pad = E_PAD - N_EDGES
    dummy_rows = jnp.arange(pad, dtype=jnp.int32) % N_NODES
    row = jnp.concatenate(
        [edge_index[0], dummy_rows]).reshape(NW, CPW, CHUNK)
    dummy_cols = N_NODES + (jnp.arange(pad, dtype=jnp.int32)
                            % (N_ACC - N_NODES))
    col_flat = jnp.concatenate([edge_index[1], dummy_cols])
    col = col_flat.reshape(NW, CPW, CHUNK)
    zeros = jnp.zeros((CHUNK, D), jnp.float32)
    acc, deg = _sc_aggregate(x, row, col, col_flat, zeros)
    out = _tc_project(acc, deg.reshape(NW, N_ACC), W, b.reshape(1, D))
    return out[:N_NODES]


# speedup vs baseline: 1.1471x; 1.1471x over previous
import jax
import jax.numpy as jnp
from jax import lax
from jax.experimental import pallas as pl
from jax.experimental.pallas import tpu as pltpu
from jax.experimental.pallas import tpu_sc as plsc

N_NODES = 10000
N_EDGES = 320000
D = 128

NC = 2
NS = 16
NW = NC * NS
L = 16
CHUNK = 64
CPW = 160
IB = 8
EPW = CPW * CHUNK
E_PAD = NW * EPW
N_ACC = 10240
RPS = N_ACC // NS
BLK_E = IB * CHUNK


def _sc_body(x_hbm, row_hbm, col_hbm, colf_hbm, zeros_hbm,
             acc_out, deg_out,
             row_v, col_v, colf_v, rows_v, deg_local,
             acc_s, sem):
    c = lax.axis_index("c")
    s = lax.axis_index("s")
    wid = s * NC + c

    pltpu.sync_copy(zeros_hbm, rows_v)

    for k in range(RPS // CHUNK):
        off = s * RPS + k * CHUNK
        pltpu.sync_copy(rows_v, acc_s.at[pl.ds(off, CHUNK)])

    zeros16 = jnp.zeros((L,), jnp.float32)

    @pl.loop(0, N_ACC // L)
    def _zero_deg(i):
        deg_local[pl.ds(i * L, L)] = zeros16

    plsc.subcore_barrier()

    ones16 = jnp.full((L,), 1.0, jnp.float32)

    for ib in range(CPW // IB):
        pltpu.sync_copy(row_hbm.at[wid, pl.ds(ib * IB, IB)], row_v)
        pltpu.sync_copy(col_hbm.at[wid, pl.ds(ib * IB, IB)], col_v)
        pltpu.sync_copy(colf_hbm.at[pl.ds(wid * EPW + ib * BLK_E, BLK_E)],
                        colf_v)

        @pl.loop(0, IB)
        def _edge_chunk(j):
            pltpu.async_copy(x_hbm.at[row_v.at[j]], rows_v, sem).wait()
            pltpu.sync_copy(rows_v, acc_s.at[col_v.at[j]], add=True)

        @pl.loop(0, BLK_E // L)
        def _deg(j):
            idx = colf_v[pl.ds(j * L, L)]
            plsc.addupdate_scatter(deg_local, [idx], ones16)

    plsc.subcore_barrier()

    out_off = s * RPS
    pltpu.sync_copy(acc_s.at[pl.ds(out_off, RPS)],
                    acc_out.at[c, pl.ds(out_off, RPS)])
    pltpu.sync_copy(deg_local, deg_out.at[pl.ds(wid * N_ACC, N_ACC)])


def _sc_aggregate(x, row_p, col_p, col_flat, zeros):
    mesh = plsc.VectorSubcoreMesh(core_axis_name="c", subcore_axis_name="s")
    return pl.kernel(
        _sc_body,
        out_type=(
            jax.ShapeDtypeStruct((NC, N_ACC, D), jnp.float32),
            jax.ShapeDtypeStruct((NW * N_ACC,), jnp.float32),
        ),
        mesh=mesh,
        compiler_params=pltpu.CompilerParams(needs_layout_passes=False),
        scratch_types=[
            pltpu.VMEM((IB, CHUNK), jnp.int32),
            pltpu.VMEM((IB, CHUNK), jnp.int32),
            pltpu.VMEM((BLK_E,), jnp.int32),
            pltpu.VMEM((CHUNK, D), jnp.float32),
            pltpu.VMEM((N_ACC,), jnp.float32),
            pltpu.VMEM_SHARED((N_ACC, D), jnp.float32),
            pltpu.SemaphoreType.DMA,
        ],
    )(x, row_p, col_p, col_flat, zeros)


BR = 1024


def _tc_body(acc_ref, deg_ref, w_ref, b_ref, o_ref):
    ssum = acc_ref[0] + acc_ref[1]
    dcol = jnp.sum(deg_ref[...], axis=0)[:, None]
    r = ssum / jnp.maximum(dcol, 1.0)
    o_ref[...] = jnp.dot(r, w_ref[...],
                         preferred_element_type=jnp.float32,
                         precision=jax.lax.Precision.HIGHEST) + b_ref[...]


def _tc_project(acc, deg, W, b2):
    return pl.pallas_call(
        _tc_body,
        grid=(N_ACC // BR,),
        in_specs=[
            pl.BlockSpec((NC, BR, D), lambda i: (0, i, 0)),
            pl.BlockSpec((NW, BR), lambda i: (0, i)),
            pl.BlockSpec((D, D), lambda i: (0, 0)),
            pl.BlockSpec((1, D), lambda i: (0, 0)),
        ],
        out_specs=pl.BlockSpec((BR, D), lambda i: (i, 0)),
        out_shape=jax.ShapeDtypeStruct((N_ACC, D), jnp.float32),
    )(acc, deg, W, b2)


@jax.jit
def kernel(x, edge_index, W, b):
    pad = E_PAD - N_EDGES
    dummy_rows = jnp.arange(pad, dtype=jnp.int32) % N_NODES
    row = jnp.concatenate(
        [edge_index[0], dummy_rows]).reshape(NW, CPW, CHUNK)
    dummy_cols = N_NODES + (jnp.arange(pad, dtype=jnp.int32)
                            % (N_ACC - N_NODES))
    col_flat = jnp.concatenate([edge_index[1], dummy_cols])
    col = col_flat.reshape(NW, CPW, CHUNK)
    zeros = jnp.zeros((CHUNK, D), jnp.float32)
    acc, deg = _sc_aggregate(x, row, col, col_flat, zeros)
    out = _tc_project(acc, deg.reshape(NW, N_ACC), W, b.reshape(1, D))
    return out[:N_NODES]
